# baseline (speedup 1.0000x reference)
import functools

import jax
import jax.numpy as jnp
from jax.experimental import pallas as pl
from jax.experimental.pallas import tpu as pltpu

N_NODES = 10000
N_EDGES = 160000


def _mm_bias_kernel(x_ref, w_ref, b_ref, o_ref):
    o_ref[...] = jnp.dot(x_ref[...], w_ref[...], precision=jax.lax.Precision.HIGHEST,
                         preferred_element_type=jnp.float32) + b_ref[...]


def _mm_bias(x, w, b, block_rows):
    M, K = x.shape
    _, C = w.shape
    assert M % block_rows == 0
    return pl.pallas_call(
        _mm_bias_kernel,
        grid=(M // block_rows,),
        in_specs=[pl.BlockSpec((block_rows, K), lambda i: (i, 0)),
                  pl.BlockSpec((K, C), lambda i: (0, 0)),
                  pl.BlockSpec((1, C), lambda i: (0, 0))],
        out_specs=pl.BlockSpec((block_rows, C), lambda i: (i, 0)),
        out_shape=jax.ShapeDtypeStruct((M, C), jnp.float32),
    )(x, w, b.reshape(1, -1))


def _prep_kernel(h_ref, pos_ref, w1h_ref, w1r_ref, b1_ref, a_ref, b_ref):
    bm = jnp.dot(pos_ref[...], w1r_ref[...], precision=jax.lax.Precision.HIGHEST, preferred_element_type=jnp.float32)
    a_ref[...] = (jnp.dot(h_ref[...], w1h_ref[...], precision=jax.lax.Precision.HIGHEST,
                          preferred_element_type=jnp.float32)
                  + bm + b1_ref[...])
    b_ref[...] = bm


def _prep(h, pos, w1h, w1r, b1, block_rows=2000):
    M, K = h.shape
    _, C = w1h.shape
    return pl.pallas_call(
        _prep_kernel,
        grid=(M // block_rows,),
        in_specs=[pl.BlockSpec((block_rows, K), lambda i: (i, 0)),
                  pl.BlockSpec((block_rows, 3), lambda i: (i, 0)),
                  pl.BlockSpec((K, C), lambda i: (0, 0)),
                  pl.BlockSpec((3, C), lambda i: (0, 0)),
                  pl.BlockSpec((1, C), lambda i: (0, 0))],
        out_specs=[pl.BlockSpec((block_rows, C), lambda i: (i, 0)),
                   pl.BlockSpec((block_rows, C), lambda i: (i, 0))],
        out_shape=[jax.ShapeDtypeStruct((M, C), jnp.float32),
                   jax.ShapeDtypeStruct((M, C), jnp.float32)],
    )(h, pos, w1h, w1r, b1.reshape(1, -1))


def _gnorm_kernel(x_ref, w_ref, b_ref, ms_ref, o_ref, *, fix_neginf, relu):
    x = x_ref[...]
    if fix_neginf:
        x = jnp.where(x == -jnp.inf, 0.0, x)
    n = x.shape[0]
    mean = jnp.sum(x, axis=0, keepdims=True) / n
    sub = x - ms_ref[...] * mean
    var = jnp.sum(sub * sub, axis=0, keepdims=True) / n
    out = w_ref[...] * sub * jax.lax.rsqrt(var + 1e-5) + b_ref[...]
    if relu:
        out = jnp.maximum(out, 0.0)
    o_ref[...] = out


def _gnorm(x, w, b, ms, fix_neginf, relu):
    M, C = x.shape
    bc = min(C, 128)
    return pl.pallas_call(
        functools.partial(_gnorm_kernel, fix_neginf=fix_neginf, relu=relu),
        grid=(C // bc,),
        in_specs=[pl.BlockSpec((M, bc), lambda j: (0, j)),
                  pl.BlockSpec((1, bc), lambda j: (0, j)),
                  pl.BlockSpec((1, bc), lambda j: (0, j)),
                  pl.BlockSpec((1, bc), lambda j: (0, j))],
        out_specs=pl.BlockSpec((M, bc), lambda j: (0, j)),
        out_shape=jax.ShapeDtypeStruct((M, C), jnp.float32),
    )(x, w.reshape(1, -1), b.reshape(1, -1), ms.reshape(1, -1))


def _edge_relu_kernel(a_ref, b_ref, o_ref):
    o_ref[...] = jnp.maximum(a_ref[...] - b_ref[...], 0.0)


def _edge_mid(ga, gb, block_rows=3200):
    M, C = ga.shape
    return pl.pallas_call(
        _edge_relu_kernel,
        grid=(M // block_rows,),
        in_specs=[pl.BlockSpec((block_rows, C), lambda i: (i, 0)),
                  pl.BlockSpec((block_rows, C), lambda i: (i, 0))],
        out_specs=pl.BlockSpec((block_rows, C), lambda i: (i, 0)),
        out_shape=jax.ShapeDtypeStruct((M, C), jnp.float32),
    )(ga, gb)


def _layer(h, pos, src, dst, W1, b1, W2, b2):
    din = W1.shape[0] - 3
    A, B = _prep(h, pos, W1[:din], W1[din:], b1)
    mid = _edge_mid(jnp.take(A, src, axis=0), jnp.take(B, dst, axis=0))
    msg = _mm_bias(mid, W2, b2, block_rows=1600)
    return jax.ops.segment_max(msg, dst, num_segments=N_NODES)


def kernel(pos, batch, edge_index, gn1_w, gn1_b, gn1_ms, gn2_w, gn2_b, gn2_ms,
           gn3_w, gn3_b, gn3_ms, gn4_w, gn4_b, gn4_ms, gn5_w, gn5_b, gn5_ms,
           gn6_w, gn6_b, gn6_ms,
           conv1_W1, conv1_b1, conv1_W2, conv1_b2,
           conv2_W1, conv2_b1, conv2_W2, conv2_b2,
           conv3_W1, conv3_b1, conv3_W2, conv3_b2,
           conv4_W1, conv4_b1, conv4_W2, conv4_b2,
           conv5_W1, conv5_b1, conv5_W2, conv5_b2,
           clf_W, clf_b):
    src = edge_index[0]
    dst = edge_index[1]
    gns = [(gn1_w, gn1_b, gn1_ms), (gn2_w, gn2_b, gn2_ms),
           (gn3_w, gn3_b, gn3_ms), (gn4_w, gn4_b, gn4_ms),
           (gn5_w, gn5_b, gn5_ms), (gn6_w, gn6_b, gn6_ms)]
    convs = [(conv1_W1, conv1_b1, conv1_W2, conv1_b2),
             (conv2_W1, conv2_b1, conv2_W2, conv2_b2),
             (conv3_W1, conv3_b1, conv3_W2, conv3_b2),
             (conv4_W1, conv4_b1, conv4_W2, conv4_b2),
             (conv5_W1, conv5_b1, conv5_W2, conv5_b2)]

    h = _gnorm(pos, *gns[0], fix_neginf=False, relu=False)
    for i in range(5):
        agg = _layer(h, pos, src, dst, *convs[i])
        h = _gnorm(agg, *gns[i + 1], fix_neginf=True, relu=True)
    return _mm_bias(h, clf_W, clf_b, block_rows=2000)


# speedup vs baseline: 1.0799x; 1.0799x over previous
import functools

import jax
import jax.numpy as jnp
from jax.experimental import pallas as pl
from jax.experimental.pallas import tpu as pltpu

N_NODES = 10000
N_EDGES = 160000


def _mm_bias_kernel(x_ref, w_ref, b_ref, o_ref):
    o_ref[...] = jnp.dot(x_ref[...], w_ref[...], precision=jax.lax.Precision.HIGHEST,
                         preferred_element_type=jnp.float32) + b_ref[...]


def _mm_bias(x, w, b, block_rows):
    M, K = x.shape
    _, C = w.shape
    assert M % block_rows == 0
    return pl.pallas_call(
        _mm_bias_kernel,
        grid=(M // block_rows,),
        in_specs=[pl.BlockSpec((block_rows, K), lambda i: (i, 0)),
                  pl.BlockSpec((K, C), lambda i: (0, 0)),
                  pl.BlockSpec((1, C), lambda i: (0, 0))],
        out_specs=pl.BlockSpec((block_rows, C), lambda i: (i, 0)),
        out_shape=jax.ShapeDtypeStruct((M, C), jnp.float32),
    )(x, w, b.reshape(1, -1))


def _prep_kernel(h_ref, pos_ref, w1h_ref, w1r_ref, b1_ref, a_ref, b_ref):
    bm = jnp.dot(pos_ref[...], w1r_ref[...], precision=jax.lax.Precision.HIGHEST, preferred_element_type=jnp.float32)
    a_ref[...] = (jnp.dot(h_ref[...], w1h_ref[...], precision=jax.lax.Precision.HIGHEST,
                          preferred_element_type=jnp.float32)
                  + bm + b1_ref[...])
    b_ref[...] = bm


def _prep(h, pos, w1h, w1r, b1, block_rows=2000):
    M, K = h.shape
    _, C = w1h.shape
    return pl.pallas_call(
        _prep_kernel,
        grid=(M // block_rows,),
        in_specs=[pl.BlockSpec((block_rows, K), lambda i: (i, 0)),
                  pl.BlockSpec((block_rows, 3), lambda i: (i, 0)),
                  pl.BlockSpec((K, C), lambda i: (0, 0)),
                  pl.BlockSpec((3, C), lambda i: (0, 0)),
                  pl.BlockSpec((1, C), lambda i: (0, 0))],
        out_specs=[pl.BlockSpec((block_rows, C), lambda i: (i, 0)),
                   pl.BlockSpec((block_rows, C), lambda i: (i, 0))],
        out_shape=[jax.ShapeDtypeStruct((M, C), jnp.float32),
                   jax.ShapeDtypeStruct((M, C), jnp.float32)],
    )(h, pos, w1h, w1r, b1.reshape(1, -1))


def _gnorm_kernel(x_ref, w_ref, b_ref, ms_ref, o_ref, *, fix_neginf, relu):
    x = x_ref[...]
    if fix_neginf:
        x = jnp.where(x == -jnp.inf, 0.0, x)
    n = x.shape[0]
    mean = jnp.sum(x, axis=0, keepdims=True) / n
    sub = x - ms_ref[...] * mean
    var = jnp.sum(sub * sub, axis=0, keepdims=True) / n
    out = w_ref[...] * sub * jax.lax.rsqrt(var + 1e-5) + b_ref[...]
    if relu:
        out = jnp.maximum(out, 0.0)
    o_ref[...] = out


def _gnorm(x, w, b, ms, fix_neginf, relu):
    M, C = x.shape
    bc = min(C, 128)
    return pl.pallas_call(
        functools.partial(_gnorm_kernel, fix_neginf=fix_neginf, relu=relu),
        grid=(C // bc,),
        in_specs=[pl.BlockSpec((M, bc), lambda j: (0, j)),
                  pl.BlockSpec((1, bc), lambda j: (0, j)),
                  pl.BlockSpec((1, bc), lambda j: (0, j)),
                  pl.BlockSpec((1, bc), lambda j: (0, j))],
        out_specs=pl.BlockSpec((M, bc), lambda j: (0, j)),
        out_shape=jax.ShapeDtypeStruct((M, C), jnp.float32),
    )(x, w.reshape(1, -1), b.reshape(1, -1), ms.reshape(1, -1))


def _msg_kernel(ga_ref, gb_ref, w_ref, b_ref, o_ref):
    mid = jnp.maximum(ga_ref[...] - gb_ref[...], 0.0)
    o_ref[...] = jnp.dot(mid, w_ref[...], precision=jax.lax.Precision.HIGHEST,
                         preferred_element_type=jnp.float32) + b_ref[...]


def _msg(ga, gb, w, b, block_rows=1600):
    M, C = ga.shape
    return pl.pallas_call(
        _msg_kernel,
        grid=(M // block_rows,),
        in_specs=[pl.BlockSpec((block_rows, C), lambda i: (i, 0)),
                  pl.BlockSpec((block_rows, C), lambda i: (i, 0)),
                  pl.BlockSpec((C, C), lambda i: (0, 0)),
                  pl.BlockSpec((1, C), lambda i: (0, 0))],
        out_specs=pl.BlockSpec((block_rows, C), lambda i: (i, 0)),
        out_shape=jax.ShapeDtypeStruct((M, C), jnp.float32),
    )(ga, gb, w, b.reshape(1, -1))


def _layer(h, pos, src_p, dst_p, W1, b1, W2, b2):
    din = W1.shape[0] - 3
    A, B = _prep(h, pos, W1[:din], W1[din:], b1)
    msg = _msg(jnp.take(A, src_p, axis=0), jnp.take(B, dst_p, axis=0), W2, b2)
    return jax.ops.segment_max(msg, dst_p, num_segments=N_NODES,
                               indices_are_sorted=True)


def kernel(pos, batch, edge_index, gn1_w, gn1_b, gn1_ms, gn2_w, gn2_b, gn2_ms,
           gn3_w, gn3_b, gn3_ms, gn4_w, gn4_b, gn4_ms, gn5_w, gn5_b, gn5_ms,
           gn6_w, gn6_b, gn6_ms,
           conv1_W1, conv1_b1, conv1_W2, conv1_b2,
           conv2_W1, conv2_b1, conv2_W2, conv2_b2,
           conv3_W1, conv3_b1, conv3_W2, conv3_b2,
           conv4_W1, conv4_b1, conv4_W2, conv4_b2,
           conv5_W1, conv5_b1, conv5_W2, conv5_b2,
           clf_W, clf_b):
    dst = edge_index[1]
    perm = jnp.argsort(dst)
    src_p = jnp.take(edge_index[0], perm)
    dst_p = jnp.take(dst, perm)
    gns = [(gn1_w, gn1_b, gn1_ms), (gn2_w, gn2_b, gn2_ms),
           (gn3_w, gn3_b, gn3_ms), (gn4_w, gn4_b, gn4_ms),
           (gn5_w, gn5_b, gn5_ms), (gn6_w, gn6_b, gn6_ms)]
    convs = [(conv1_W1, conv1_b1, conv1_W2, conv1_b2),
             (conv2_W1, conv2_b1, conv2_W2, conv2_b2),
             (conv3_W1, conv3_b1, conv3_W2, conv3_b2),
             (conv4_W1, conv4_b1, conv4_W2, conv4_b2),
             (conv5_W1, conv5_b1, conv5_W2, conv5_b2)]

    h = _gnorm(pos, *gns[0], fix_neginf=False, relu=False)
    for i in range(5):
        agg = _layer(h, pos, src_p, dst_p, *convs[i])
        h = _gnorm(agg, *gns[i + 1], fix_neginf=True, relu=True)
    return _mm_bias(h, clf_W, clf_b, block_rows=2000)
